# baseline (speedup 1.0000x reference)
import jax
import jax.numpy as jnp
from jax import lax
from jax.experimental import pallas as pl
from jax.experimental.pallas import tpu as pltpu
from jax.experimental.pallas import tpu_sc as plsc

N = 10000
E = 320000
D = 128

NC = 2
NS = 16
DH = D // NC
EPT = E // NS
K = 80
NB = EPT // K
NP = 10240
RB = 128
RPT = NP // NS
RCH = RPT // RB
LJ = DH // 16


def _scale_batch(rows, adj_v, b):
    for g in range(K // 16):
        av = adj_v[b, pl.ds(g * 16, 16)]
        for l in range(16):
            a = av[l]
            for j in range(LJ):
                sl = pl.ds(j * 16, 16)
                e = g * 16 + l
                rows[e, sl] = rows[e, sl] * a


def _sc_body(emb_hbm, src_hbm, dst_hbm, adj_hbm, out_hbm,
             src_v, dst_v, adj_v, rows, rows_b, sem, sem_b, acc):
    c = lax.axis_index("c")
    s = lax.axis_index("s")

    pltpu.sync_copy(src_hbm.at[s], src_v)
    pltpu.sync_copy(dst_hbm.at[s], dst_v)
    pltpu.sync_copy(adj_hbm.at[s], adj_v)

    @pl.loop(0, NB)
    def _(b):
        for g in range(K // 16):
            sl = pl.ds(g * 16, 16)
            sv = src_v[b, sl]
            src_v[b, sl] = sv + sv + c

    zeros16 = jnp.zeros((16,), jnp.float32)

    @pl.loop(0, RB)
    def _(e):
        for j in range(LJ):
            rows[e, pl.ds(j * 16, 16)] = zeros16

    for z in range(RCH):
        pltpu.sync_copy(rows, acc.at[pl.ds(s * RPT + z * RB, RB)])
    plsc.subcore_barrier()

    rows_a = rows.at[pl.ds(0, K)]
    pltpu.async_copy(emb_hbm.at[src_v.at[0]], rows_a, sem)

    @pl.loop(0, NB // 2)
    def _(h):
        b0 = h * 2
        b1 = b0 + 1
        pltpu.async_copy(emb_hbm.at[src_v.at[b1]], rows_b, sem_b)
        pltpu.make_async_copy(emb_hbm.at[src_v.at[b0]], rows_a, sem).wait()
        _scale_batch(rows, adj_v, b0)
        pltpu.sync_copy(rows_a, acc.at[dst_v.at[b0]], add=True)

        b2 = b0 + 2

        @pl.when(b2 < NB)
        def _():
            pltpu.async_copy(emb_hbm.at[src_v.at[b2]], rows_a, sem)

        pltpu.make_async_copy(emb_hbm.at[src_v.at[b1]], rows_b, sem_b).wait()
        _scale_batch(rows_b, adj_v, b1)
        pltpu.sync_copy(rows_b, acc.at[dst_v.at[b1]], add=True)

    plsc.subcore_barrier()

    for z in range(RCH):
        r0 = s * RPT + z * RB
        pltpu.sync_copy(acc.at[pl.ds(r0, RB)], rows)
        pltpu.sync_copy(rows, out_hbm.at[c, pl.ds(r0, RB)])


@jax.jit
def _sc_aggregate(emb2, src, dst, adj):
    mesh = plsc.VectorSubcoreMesh(core_axis_name="c", subcore_axis_name="s")
    f = pl.kernel(
        _sc_body,
        out_type=jax.ShapeDtypeStruct((NC, NP, DH), jnp.float32),
        mesh=mesh,
        compiler_params=pltpu.CompilerParams(use_tc_tiling_on_sc=False),
        scratch_types=[
            pltpu.VMEM((NB, K), jnp.int32),
            pltpu.VMEM((NB, K), jnp.int32),
            pltpu.VMEM((NB, K), jnp.float32),
            pltpu.VMEM((RB, DH), jnp.float32),
            pltpu.VMEM((K, DH), jnp.float32),
            pltpu.SemaphoreType.DMA,
            pltpu.SemaphoreType.DMA,
            pltpu.VMEM_SHARED((NP, DH), jnp.float32),
        ],
    )
    return f(emb2, src, dst, adj)


def _tc_body(p_ref, e_ref, w1_ref, w2_ref, o_ref):
    r_lo = p_ref[0]
    r_hi = p_ref[1]
    e = e_ref[...]
    w1 = w1_ref[...]
    w2 = w2_ref[...]
    f32 = jnp.float32
    o_ref[...] = (
        jnp.dot(r_lo, w1[:DH], preferred_element_type=f32)
        + jnp.dot(r_hi, w1[DH:], preferred_element_type=f32)
        + jnp.dot(r_lo * e[:, :DH], w2[:DH], preferred_element_type=f32)
        + jnp.dot(r_hi * e[:, DH:], w2[DH:], preferred_element_type=f32)
    )


@jax.jit
def _tc_transform(partials, embeddings, W1, W2):
    BN = 1000
    return pl.pallas_call(
        _tc_body,
        grid=(N // BN,),
        in_specs=[
            pl.BlockSpec((NC, BN, DH), lambda i: (0, i, 0)),
            pl.BlockSpec((BN, D), lambda i: (i, 0)),
            pl.BlockSpec((D, D), lambda i: (0, 0)),
            pl.BlockSpec((D, D), lambda i: (0, 0)),
        ],
        out_specs=pl.BlockSpec((BN, D), lambda i: (i, 0)),
        out_shape=jax.ShapeDtypeStruct((N, D), jnp.float32),
    )(partials, embeddings, W1, W2)


def kernel(embeddings, edge_index, adj_values, W1, W2):
    emb2 = embeddings.reshape(N * NC, DH)
    dst = edge_index[0].reshape(NS, NB, K)
    src = edge_index[1].reshape(NS, NB, K)
    adj = adj_values.reshape(NS, NB, K)
    partials = _sc_aggregate(emb2, src, dst, adj)
    return _tc_transform(partials, embeddings, W1, W2)


# speedup vs baseline: 1.1403x; 1.1403x over previous
import jax
import jax.numpy as jnp
from jax import lax
from jax.experimental import pallas as pl
from jax.experimental.pallas import tpu as pltpu
from jax.experimental.pallas import tpu_sc as plsc

N = 10000
E = 320000
D = 128

NC = 2
NS = 16
DH = D // NC
EPT = E // NS
K = 80
NB = EPT // K
NP = 10240
RPT = NP // NS
RCH = RPT // K
LJ = DH // 16


def _scale_batch(rows, adj_v, b):
    @plsc.parallel_loop(0, K // 16)
    def _(g):
        av = adj_v[b, pl.ds(g * 16, 16)]
        for l in range(16):
            a = av[l]
            for j in range(LJ):
                sl = pl.ds(j * 16, 16)
                rows[g * 16 + l, sl] = rows[g * 16 + l, sl] * a


def _sc_body(emb_hbm, src_hbm, dst_hbm, adj_hbm, out_hbm,
             src_v, dst_v, adj_v, r0, r1, r2, r3,
             g0, g1, g2, g3, s0, s1, s2, s3, acc):
    c = lax.axis_index("c")
    s = lax.axis_index("s")
    bufs = (r0, r1, r2, r3)
    gsems = (g0, g1, g2, g3)
    ssems = (s0, s1, s2, s3)

    pltpu.sync_copy(src_hbm.at[s], src_v)
    pltpu.sync_copy(dst_hbm.at[s], dst_v)
    pltpu.sync_copy(adj_hbm.at[s], adj_v)

    @pl.loop(0, NB)
    def _(b):
        for g in range(K // 16):
            sl = pl.ds(g * 16, 16)
            sv = src_v[b, sl]
            src_v[b, sl] = sv + sv + c

    zeros16 = jnp.zeros((16,), jnp.float32)

    @pl.loop(0, K)
    def _(e):
        for j in range(LJ):
            r0[e, pl.ds(j * 16, 16)] = zeros16

    for z in range(RCH):
        pltpu.sync_copy(r0, acc.at[pl.ds(s * RPT + z * K, K)])
    plsc.subcore_barrier()

    def start_g(b, u):
        pltpu.async_copy(emb_hbm.at[src_v.at[b]], bufs[u], gsems[u])

    def wait_g(b, u):
        pltpu.make_async_copy(emb_hbm.at[src_v.at[b]], bufs[u], gsems[u]).wait()

    def start_s(b, u):
        pltpu.async_copy(bufs[u], acc.at[dst_v.at[b]], ssems[u], add=True)

    def wait_s(b, u):
        pltpu.make_async_copy(bufs[u], acc.at[dst_v.at[b]], ssems[u]).wait()

    def phase(m, u):
        wait_g(m, u)
        _scale_batch(bufs[u], adj_v, m)
        start_s(m, u)
        q = m + 2
        uq = (u + 2) % 4

        @pl.when(q < NB)
        def _():
            wait_s(q - 4, uq)
            start_g(q, uq)

    start_g(0, 0)
    start_g(1, 1)
    wait_g(0, 0)
    _scale_batch(r0, adj_v, 0)
    start_s(0, 0)
    start_g(2, 2)
    wait_g(1, 1)
    _scale_batch(r1, adj_v, 1)
    start_s(1, 1)
    start_g(3, 3)

    @pl.loop(0, (NB - 2) // 4)
    def _(jj):
        m0 = 2 + jj * 4
        for u_off in range(4):
            phase(m0 + u_off, (2 + u_off) % 4)

    wait_s(NB - 2, (NB - 2) % 4)
    wait_s(NB - 1, (NB - 1) % 4)
    plsc.subcore_barrier()

    for z in range(RCH):
        rr = s * RPT + z * K
        pltpu.sync_copy(acc.at[pl.ds(rr, K)], r0)
        pltpu.sync_copy(r0, out_hbm.at[c, pl.ds(rr, K)])


@jax.jit
def _sc_aggregate(emb2, src, dst, adj):
    mesh = plsc.VectorSubcoreMesh(core_axis_name="c", subcore_axis_name="s")
    f = pl.kernel(
        _sc_body,
        out_type=jax.ShapeDtypeStruct((NC, NP, DH), jnp.float32),
        mesh=mesh,
        compiler_params=pltpu.CompilerParams(use_tc_tiling_on_sc=False),
        scratch_types=[
            pltpu.VMEM((NB, K), jnp.int32),
            pltpu.VMEM((NB, K), jnp.int32),
            pltpu.VMEM((NB, K), jnp.float32),
            pltpu.VMEM((K, DH), jnp.float32),
            pltpu.VMEM((K, DH), jnp.float32),
            pltpu.VMEM((K, DH), jnp.float32),
            pltpu.VMEM((K, DH), jnp.float32),
            pltpu.SemaphoreType.DMA,
            pltpu.SemaphoreType.DMA,
            pltpu.SemaphoreType.DMA,
            pltpu.SemaphoreType.DMA,
            pltpu.SemaphoreType.DMA,
            pltpu.SemaphoreType.DMA,
            pltpu.SemaphoreType.DMA,
            pltpu.SemaphoreType.DMA,
            pltpu.VMEM_SHARED((NP, DH), jnp.float32),
        ],
    )
    return f(emb2, src, dst, adj)


def _tc_body(p_ref, e_ref, w1_ref, w2_ref, o_ref):
    r_lo = p_ref[0]
    r_hi = p_ref[1]
    e = e_ref[...]
    w1 = w1_ref[...]
    w2 = w2_ref[...]
    f32 = jnp.float32
    o_ref[...] = (
        jnp.dot(r_lo, w1[:DH], preferred_element_type=f32)
        + jnp.dot(r_hi, w1[DH:], preferred_element_type=f32)
        + jnp.dot(r_lo * e[:, :DH], w2[:DH], preferred_element_type=f32)
        + jnp.dot(r_hi * e[:, DH:], w2[DH:], preferred_element_type=f32)
    )


@jax.jit
def _tc_transform(partials, embeddings, W1, W2):
    BN = 1000
    return pl.pallas_call(
        _tc_body,
        grid=(N // BN,),
        in_specs=[
            pl.BlockSpec((NC, BN, DH), lambda i: (0, i, 0)),
            pl.BlockSpec((BN, D), lambda i: (i, 0)),
            pl.BlockSpec((D, D), lambda i: (0, 0)),
            pl.BlockSpec((D, D), lambda i: (0, 0)),
        ],
        out_specs=pl.BlockSpec((BN, D), lambda i: (i, 0)),
        out_shape=jax.ShapeDtypeStruct((N, D), jnp.float32),
    )(partials, embeddings, W1, W2)


def kernel(embeddings, edge_index, adj_values, W1, W2):
    emb2 = embeddings.reshape(N * NC, DH)
    dst = edge_index[0].reshape(NS, NB, K)
    src = edge_index[1].reshape(NS, NB, K)
    adj = adj_values.reshape(NS, NB, K)
    partials = _sc_aggregate(emb2, src, dst, adj)
    return _tc_transform(partials, embeddings, W1, W2)
